# baseline (speedup 1.0000x reference)
import functools
import jax
import jax.numpy as jnp
from jax.experimental import pallas as pl
from jax.experimental.pallas import tpu as pltpu

FP8_MAX = 448.0
BLK = 128


def _act_qdq_kernel(x_ref, o_ref):
    k = x_ref.shape[1]
    for kb in range(k // BLK):
        sl = slice(kb * BLK, (kb + 1) * BLK)
        g = x_ref[:, sl].astype(jnp.float32)
        amax = jnp.max(jnp.abs(g), axis=1, keepdims=True)
        scale = jnp.maximum(amax, 1e-12) / FP8_MAX
        q = (g * (1.0 / scale)).astype(jnp.float8_e4m3fn).astype(jnp.float32)
        o_ref[:, sl] = (q * scale).astype(jnp.bfloat16)


def _gemm_wdq_kernel(s_ref, x_ref, w_ref, o_ref, wdq_ref, *, bn, k, ck):
    j = pl.program_id(1)
    nb = bn // BLK
    row0 = j * nb
    nchunk = k // ck
    ckb = ck // BLK
    acc = None
    for c in range(nchunk):
        buf = c % 2
        for i in range(nb):
            rs = slice(i * BLK, (i + 1) * BLK)
            for kb in range(ckb):
                gkb = c * ckb + kb
                wv = w_ref[rs, gkb * BLK:(gkb + 1) * BLK].astype(jnp.bfloat16)
                s = s_ref[row0 + i, gkb].astype(jnp.bfloat16)
                wdq_ref[buf, rs, kb * BLK:(kb + 1) * BLK] = wv * s
        d = jax.lax.dot_general(
            x_ref[:, c * ck:(c + 1) * ck], wdq_ref[buf],
            dimension_numbers=(((1,), (1,)), ((), ())),
            preferred_element_type=jnp.float32,
        )
        acc = d if acc is None else acc + d
    o_ref[...] = acc.astype(jnp.bfloat16)


@jax.jit
def kernel(input, weight_fp8, weight_scale):
    m, k = input.shape
    n = weight_fp8.shape[0]

    bmq = 512
    x_dq = pl.pallas_call(
        _act_qdq_kernel,
        grid=(m // bmq,),
        in_specs=[pl.BlockSpec((bmq, k), lambda i: (i, 0))],
        out_specs=pl.BlockSpec((bmq, k), lambda i: (i, 0)),
        out_shape=jax.ShapeDtypeStruct((m, k), jnp.bfloat16),
        compiler_params=pltpu.CompilerParams(
            dimension_semantics=("parallel",),
        ),
    )(input)

    wq8 = weight_fp8.astype(jnp.float8_e4m3fn)

    bm, bn, ck = 1024, 1024, 1024
    out = pl.pallas_call(
        functools.partial(_gemm_wdq_kernel, bn=bn, k=k, ck=ck),
        grid=(m // bm, n // bn),
        in_specs=[
            pl.BlockSpec(memory_space=pltpu.SMEM),
            pl.BlockSpec((bm, k), lambda i, j: (i, 0)),
            pl.BlockSpec((bn, k), lambda i, j: (j, 0)),
        ],
        out_specs=pl.BlockSpec((bm, bn), lambda i, j: (i, j)),
        out_shape=jax.ShapeDtypeStruct((m, n), jnp.bfloat16),
        scratch_shapes=[
            pltpu.VMEM((2, bn, ck), jnp.bfloat16),
        ],
        compiler_params=pltpu.CompilerParams(
            dimension_semantics=("parallel", "arbitrary"),
            vmem_limit_bytes=56 * 1024 * 1024,
        ),
    )(weight_scale, x_dq, wq8)
    return out


# speedup vs baseline: 1.0021x; 1.0021x over previous
import functools
import jax
import jax.numpy as jnp
from jax.experimental import pallas as pl
from jax.experimental.pallas import tpu as pltpu

FP8_MAX = 448.0
BLK = 128


def _act_qdq_kernel(x_ref, o_ref):
    k = x_ref.shape[1]
    for kb in range(k // BLK):
        sl = slice(kb * BLK, (kb + 1) * BLK)
        g = x_ref[:, sl].astype(jnp.float32)
        amax = jnp.max(jnp.abs(g), axis=1, keepdims=True)
        scale = jnp.maximum(amax, 1e-12) / FP8_MAX
        q = (g * (1.0 / scale)).astype(jnp.float8_e4m3fn).astype(jnp.float32)
        o_ref[:, sl] = (q * scale).astype(jnp.bfloat16)


def _gemm_wdq_kernel(s_ref, x_ref, w_ref, o_ref, wdq_ref, *, bn, k, ck):
    j = pl.program_id(1)
    nb = bn // BLK
    row0 = j * nb
    nchunk = k // ck
    ckb = ck // BLK
    acc = None
    for c in range(nchunk):
        buf = c % 2
        for i in range(nb):
            rs = slice(i * BLK, (i + 1) * BLK)
            for kb in range(ckb):
                gkb = c * ckb + kb
                wv = w_ref[rs, gkb * BLK:(gkb + 1) * BLK].astype(jnp.bfloat16)
                s = s_ref[row0 + i, gkb].astype(jnp.bfloat16)
                wdq_ref[buf, rs, kb * BLK:(kb + 1) * BLK] = wv * s
        d = jax.lax.dot_general(
            x_ref[:, c * ck:(c + 1) * ck], wdq_ref[buf],
            dimension_numbers=(((1,), (1,)), ((), ())),
            preferred_element_type=jnp.float32,
        )
        acc = d if acc is None else acc + d
    o_ref[...] = acc.astype(jnp.bfloat16)


@jax.jit
def kernel(input, weight_fp8, weight_scale):
    m, k = input.shape
    n = weight_fp8.shape[0]

    bmq = 512
    x_dq = pl.pallas_call(
        _act_qdq_kernel,
        grid=(m // bmq,),
        in_specs=[pl.BlockSpec((bmq, k), lambda i: (i, 0))],
        out_specs=pl.BlockSpec((bmq, k), lambda i: (i, 0)),
        out_shape=jax.ShapeDtypeStruct((m, k), jnp.bfloat16),
        compiler_params=pltpu.CompilerParams(
            dimension_semantics=("parallel",),
        ),
    )(input)

    wq8 = weight_fp8.astype(jnp.float8_e4m3fn)

    bm, bn, ck = 1024, 1024, 512
    out = pl.pallas_call(
        functools.partial(_gemm_wdq_kernel, bn=bn, k=k, ck=ck),
        grid=(m // bm, n // bn),
        in_specs=[
            pl.BlockSpec(memory_space=pltpu.SMEM),
            pl.BlockSpec((bm, k), lambda i, j: (i, 0)),
            pl.BlockSpec((bn, k), lambda i, j: (j, 0)),
        ],
        out_specs=pl.BlockSpec((bm, bn), lambda i, j: (i, j)),
        out_shape=jax.ShapeDtypeStruct((m, n), jnp.bfloat16),
        scratch_shapes=[
            pltpu.VMEM((2, bn, ck), jnp.bfloat16),
        ],
        compiler_params=pltpu.CompilerParams(
            dimension_semantics=("parallel", "arbitrary"),
            vmem_limit_bytes=56 * 1024 * 1024,
        ),
    )(weight_scale, x_dq, wq8)
    return out
